# baseline (speedup 1.0000x reference)
import functools
import math

import jax
import jax.numpy as jnp
from jax import lax
from jax.experimental import pallas as pl
from jax.experimental.pallas import tpu as pltpu
from jax.experimental.pallas import tpu_sc as plsc

_NUM_CLASSES = 100000
_NUM_SAMPLED = 4096
_HIDDEN = 1024
_N = 2 * 2048
_K = _N + _NUM_SAMPLED
_S = _NUM_SAMPLED
_LOG_DENOM = math.log(_NUM_CLASSES + 1.0)

_NW = 32
_KW = _K // _NW
_GCHUNK = 32
_NCHUNK = _KW // _GCHUNK


def _sc_gather(weights, cls, bias):
    mesh = plsc.VectorSubcoreMesh(core_axis_name="c", subcore_axis_name="s")

    @functools.partial(
        pl.kernel,
        mesh=mesh,
        out_type=[
            jax.ShapeDtypeStruct((_K, _HIDDEN), jnp.float32),
            jax.ShapeDtypeStruct((_NW * _NCHUNK, _GCHUNK), jnp.float32),
        ],
        scratch_types=[
            pltpu.VMEM((_NCHUNK, _GCHUNK), jnp.int32),
            pltpu.VMEM((_GCHUNK, _HIDDEN), jnp.float32),
            pltpu.VMEM((_GCHUNK, _HIDDEN), jnp.float32),
            pltpu.VMEM((_NCHUNK, _GCHUNK), jnp.float32),
            pltpu.SemaphoreType.DMA,
            pltpu.SemaphoreType.DMA,
            pltpu.SemaphoreType.DMA,
        ],
    )
    def gather_kernel(w_hbm, cls_hbm, bias_hbm,
                      out_hbm, biasg_hbm,
                      cls_v, row_a, row_b, biasg_v, gsem, wsem, bsem):
        wid = lax.axis_index("s") * 2 + lax.axis_index("c")
        base_k = wid * _KW

        pltpu.sync_copy(cls_hbm.at[wid], cls_v)

        bufs = (row_a, row_b)
        gathers = [None] * _NCHUNK
        writes = [None] * _NCHUNK
        bias_gathers = [None] * _NCHUNK
        for i in range(_NCHUNK):
            if i >= 2:
                writes[i - 2].wait()
            gathers[i] = pltpu.async_copy(
                w_hbm.at[cls_v.at[i]], bufs[i % 2], gsem)
            bias_gathers[i] = pltpu.async_copy(
                bias_hbm.at[cls_v.at[i]], biasg_v.at[i], bsem)
            if i >= 1:
                gathers[i - 1].wait()
                writes[i - 1] = pltpu.async_copy(
                    bufs[(i - 1) % 2],
                    out_hbm.at[pl.ds(base_k + (i - 1) * _GCHUNK, _GCHUNK)],
                    wsem)
        gathers[_NCHUNK - 1].wait()
        writes[_NCHUNK - 1] = pltpu.async_copy(
            bufs[(_NCHUNK - 1) % 2],
            out_hbm.at[pl.ds(base_k + (_NCHUNK - 1) * _GCHUNK, _GCHUNK)],
            wsem)
        writes[_NCHUNK - 2].wait()
        writes[_NCHUNK - 1].wait()
        for i in range(_NCHUNK):
            bias_gathers[i].wait()
        pltpu.sync_copy(biasg_v, biasg_hbm.at[pl.ds(wid * _NCHUNK, _NCHUNK)])

    return gather_kernel(weights, cls, bias)


def _log_corr(cf):
    return jnp.log(_NUM_SAMPLED * jnp.log((cf + 2.0) / (cf + 1.0)) / _LOG_DENOM)


_SCK = 1024


def _loss_body(x_ref, tw_ref, sw_ref, bt_ref, bs_ref, lab_ref, samp_ref, out_ref):
    xb = x_ref[...]
    tw = tw_ref[...]
    labels = lab_ref[...]
    bias_t = bt_ref[...]

    xb16 = xb.astype(jnp.bfloat16)

    def chunk(j, carry):
        m, se = carry
        sw = sw_ref[pl.ds(j * _SCK, _SCK), :]
        samp_j = samp_ref[0, pl.ds(j * _SCK, _SCK)][None, :]
        lg = lax.dot_general(
            xb16, sw.astype(jnp.bfloat16),
            dimension_numbers=(((1,), (1,)), ((), ())),
            preferred_element_type=jnp.float32)
        lg = (lg + bs_ref[0, pl.ds(j * _SCK, _SCK)][None, :]
              - _log_corr(samp_j.astype(jnp.float32)))
        lg = jnp.where(labels == samp_j, -1e9, lg)
        m_new = jnp.maximum(m, jnp.max(lg, axis=1, keepdims=True))
        se = (se * jnp.exp(m - m_new)
              + jnp.sum(jnp.exp(lg - m_new), axis=1, keepdims=True))
        return m_new, se

    m0 = jnp.full((xb.shape[0], 1), -1e30, jnp.float32)
    se0 = jnp.zeros((xb.shape[0], 1), jnp.float32)
    m, sumexp = lax.fori_loop(0, _S // _SCK, chunk, (m0, se0))

    true_logits = (jnp.sum(xb * tw, axis=1, keepdims=True)
                   + bias_t - _log_corr(labels.astype(jnp.float32)))
    big = jnp.maximum(m, true_logits)
    sumexp = sumexp * jnp.exp(m - big) + jnp.exp(true_logits - big)
    out_ref[...] = jnp.log(sumexp) + big - true_logits


_BN = 512


def _tc_loss(x2, w2, bt, bs, lab2, samp2):
    return pl.pallas_call(
        _loss_body,
        grid=(_N // _BN,),
        in_specs=[
            pl.BlockSpec((_BN, _HIDDEN), lambda i: (i, 0)),
            pl.BlockSpec((_BN, _HIDDEN), lambda i: (i, 0)),
            pl.BlockSpec((_S, _HIDDEN), lambda i: (_N // _S, 0)),
            pl.BlockSpec((_BN, 1), lambda i: (i, 0)),
            pl.BlockSpec((1, _S), lambda i: (0, 0)),
            pl.BlockSpec((_BN, 1), lambda i: (i, 0)),
            pl.BlockSpec((1, _S), lambda i: (0, 0)),
        ],
        out_specs=pl.BlockSpec((_BN, 1), lambda i: (i, 0)),
        out_shape=jax.ShapeDtypeStruct((_N, 1), jnp.float32),
    )(x2, w2, w2, bt, bs, lab2, samp2)


def kernel(y_true, input, projection, bias, sampled):
    labels = y_true.reshape(-1)
    x2 = input.reshape(_N, _HIDDEN)
    cls = jnp.concatenate([labels, sampled])
    weights = jnp.swapaxes(projection, 0, 1)
    w2, bias_g = _sc_gather(weights, cls.reshape(_NW, _NCHUNK, _GCHUNK), bias)
    bias_g = bias_g.reshape(-1)
    loss = _tc_loss(x2, w2,
                    bias_g[:_N].reshape(_N, 1), bias_g[_N:].reshape(1, _S),
                    labels.reshape(_N, 1), sampled.reshape(1, _S))
    return loss.reshape(-1)


# speedup vs baseline: 1.1050x; 1.1050x over previous
import functools
import math

import jax
import jax.numpy as jnp
from jax import lax
from jax.experimental import pallas as pl
from jax.experimental.pallas import tpu as pltpu
from jax.experimental.pallas import tpu_sc as plsc

_NUM_CLASSES = 100000
_NUM_SAMPLED = 4096
_HIDDEN = 1024
_N = 2 * 2048
_K = _N + _NUM_SAMPLED
_S = _NUM_SAMPLED
_LOG_DENOM = math.log(_NUM_CLASSES + 1.0)

_NW = 32
_KW = _K // _NW
_GCHUNK = 32
_NCHUNK = _KW // _GCHUNK


def _sc_gather(weights, cls, bias):
    mesh = plsc.VectorSubcoreMesh(core_axis_name="c", subcore_axis_name="s")

    @functools.partial(
        pl.kernel,
        mesh=mesh,
        out_type=[
            jax.ShapeDtypeStruct((_K, _HIDDEN), jnp.float32),
            jax.ShapeDtypeStruct((_NW * _NCHUNK, _GCHUNK), jnp.float32),
        ],
        scratch_types=[
            pltpu.VMEM((_NCHUNK, _GCHUNK), jnp.int32),
            pltpu.VMEM((_GCHUNK, _HIDDEN), jnp.float32),
            pltpu.VMEM((_GCHUNK, _HIDDEN), jnp.float32),
            pltpu.VMEM((_NCHUNK, _GCHUNK), jnp.float32),
            pltpu.SemaphoreType.DMA,
            pltpu.SemaphoreType.DMA,
            pltpu.SemaphoreType.DMA,
        ],
    )
    def gather_kernel(w_hbm, cls_hbm, bias_hbm,
                      out_hbm, biasg_hbm,
                      cls_v, row_a, row_b, biasg_v, gsem, wsem, bsem):
        wid = lax.axis_index("s") * 2 + lax.axis_index("c")
        base_k = wid * _KW

        pltpu.sync_copy(cls_hbm.at[wid], cls_v)

        bufs = (row_a, row_b)
        gathers = [None] * _NCHUNK
        writes = [None] * _NCHUNK
        bias_gathers = [None] * _NCHUNK
        for i in range(_NCHUNK):
            if i >= 2:
                writes[i - 2].wait()
            gathers[i] = pltpu.async_copy(
                w_hbm.at[cls_v.at[i]], bufs[i % 2], gsem)
            bias_gathers[i] = pltpu.async_copy(
                bias_hbm.at[cls_v.at[i]], biasg_v.at[i], bsem)
            if i >= 1:
                gathers[i - 1].wait()
                writes[i - 1] = pltpu.async_copy(
                    bufs[(i - 1) % 2],
                    out_hbm.at[pl.ds(base_k + (i - 1) * _GCHUNK, _GCHUNK)],
                    wsem)
        gathers[_NCHUNK - 1].wait()
        writes[_NCHUNK - 1] = pltpu.async_copy(
            bufs[(_NCHUNK - 1) % 2],
            out_hbm.at[pl.ds(base_k + (_NCHUNK - 1) * _GCHUNK, _GCHUNK)],
            wsem)
        writes[_NCHUNK - 2].wait()
        writes[_NCHUNK - 1].wait()
        for i in range(_NCHUNK):
            bias_gathers[i].wait()
        pltpu.sync_copy(biasg_v, biasg_hbm.at[pl.ds(wid * _NCHUNK, _NCHUNK)])

    return gather_kernel(weights, cls, bias)


def _log_corr(cf):
    return jnp.log(_NUM_SAMPLED * jnp.log((cf + 2.0) / (cf + 1.0)) / _LOG_DENOM)


_SCK = 1024


def _loss_body(x_ref, tw_ref, sw_ref, bt_ref, bs_ref, lab_ref, samp_ref, out_ref):
    xb = x_ref[...]
    tw = tw_ref[...]
    labels = lab_ref[...]
    bias_t = bt_ref[...]

    sampled = samp_ref[...]
    lg = lax.dot_general(
        xb.astype(jnp.bfloat16), sw_ref[...].astype(jnp.bfloat16),
        dimension_numbers=(((1,), (1,)), ((), ())),
        preferred_element_type=jnp.float32)
    lg = lg + bs_ref[...] - _log_corr(sampled.astype(jnp.float32))
    lg = jnp.where(labels == sampled, -1e9, lg)

    true_logits = (jnp.sum(xb * tw, axis=1, keepdims=True)
                   + bias_t - _log_corr(labels.astype(jnp.float32)))
    m = jnp.maximum(jnp.max(lg, axis=1, keepdims=True), true_logits)
    sumexp = (jnp.sum(jnp.exp(lg - m), axis=1, keepdims=True)
              + jnp.exp(true_logits - m))
    out_ref[...] = jnp.log(sumexp) + m - true_logits


_BN = 512


def _tc_loss(x2, w2, bt, bs, lab2, samp2):
    return pl.pallas_call(
        _loss_body,
        grid=(_N // _BN,),
        in_specs=[
            pl.BlockSpec((_BN, _HIDDEN), lambda i: (i, 0)),
            pl.BlockSpec((_BN, _HIDDEN), lambda i: (i, 0)),
            pl.BlockSpec((_S, _HIDDEN), lambda i: (_N // _S, 0)),
            pl.BlockSpec((_BN, 1), lambda i: (i, 0)),
            pl.BlockSpec((1, _S), lambda i: (0, 0)),
            pl.BlockSpec((_BN, 1), lambda i: (i, 0)),
            pl.BlockSpec((1, _S), lambda i: (0, 0)),
        ],
        out_specs=pl.BlockSpec((_BN, 1), lambda i: (i, 0)),
        out_shape=jax.ShapeDtypeStruct((_N, 1), jnp.float32),
    )(x2, w2, w2, bt, bs, lab2, samp2)


def kernel(y_true, input, projection, bias, sampled):
    labels = y_true.reshape(-1)
    x2 = input.reshape(_N, _HIDDEN)
    cls = jnp.concatenate([labels, sampled])
    weights = jnp.swapaxes(projection, 0, 1)
    w2, bias_g = _sc_gather(weights, cls.reshape(_NW, _NCHUNK, _GCHUNK), bias)
    bias_g = bias_g.reshape(-1)
    loss = _tc_loss(x2, w2,
                    bias_g[:_N].reshape(_N, 1), bias_g[_N:].reshape(1, _S),
                    labels.reshape(_N, 1), sampled.reshape(1, _S))
    return loss.reshape(-1)


# speedup vs baseline: 1.1050x; 1.0000x over previous
import functools
import math

import jax
import jax.numpy as jnp
from jax import lax
from jax.experimental import pallas as pl
from jax.experimental.pallas import tpu as pltpu
from jax.experimental.pallas import tpu_sc as plsc

_NUM_CLASSES = 100000
_NUM_SAMPLED = 4096
_HIDDEN = 1024
_N = 2 * 2048
_K = _N + _NUM_SAMPLED
_S = _NUM_SAMPLED
_LOG_DENOM = math.log(_NUM_CLASSES + 1.0)

_NW = 32
_KW = _K // _NW
_GCHUNK = 32
_NCHUNK = _KW // _GCHUNK


def _sc_gather(weights, cls, bias):
    mesh = plsc.VectorSubcoreMesh(core_axis_name="c", subcore_axis_name="s")

    @functools.partial(
        pl.kernel,
        mesh=mesh,
        out_type=[
            jax.ShapeDtypeStruct((_K, _HIDDEN), jnp.float32),
            jax.ShapeDtypeStruct((_NW * _NCHUNK, _GCHUNK), jnp.float32),
        ],
        scratch_types=[
            pltpu.VMEM((_NCHUNK, _GCHUNK), jnp.int32),
            pltpu.VMEM((_GCHUNK, _HIDDEN), jnp.float32),
            pltpu.VMEM((_GCHUNK, _HIDDEN), jnp.float32),
            pltpu.VMEM((_NCHUNK, _GCHUNK), jnp.float32),
            pltpu.SemaphoreType.DMA,
            pltpu.SemaphoreType.DMA,
            pltpu.SemaphoreType.DMA,
        ],
    )
    def gather_kernel(w_hbm, cls_hbm, bias_hbm,
                      out_hbm, biasg_hbm,
                      cls_v, row_a, row_b, biasg_v, gsem, wsem, bsem):
        wid = lax.axis_index("s") * 2 + lax.axis_index("c")
        base_k = wid * _KW

        pltpu.sync_copy(cls_hbm.at[wid], cls_v)

        bufs = (row_a, row_b)
        gathers = [None] * _NCHUNK
        writes = [None] * _NCHUNK
        bias_gathers = [None] * _NCHUNK
        for i in range(_NCHUNK):
            if i >= 2:
                writes[i - 2].wait()
            gathers[i] = pltpu.async_copy(
                w_hbm.at[cls_v.at[i]], bufs[i % 2], gsem)
            bias_gathers[i] = pltpu.async_copy(
                bias_hbm.at[cls_v.at[i]], biasg_v.at[i], bsem)
            if i >= 1:
                gathers[i - 1].wait()
                writes[i - 1] = pltpu.async_copy(
                    bufs[(i - 1) % 2],
                    out_hbm.at[pl.ds(base_k + (i - 1) * _GCHUNK, _GCHUNK)],
                    wsem)
        gathers[_NCHUNK - 1].wait()
        writes[_NCHUNK - 1] = pltpu.async_copy(
            bufs[(_NCHUNK - 1) % 2],
            out_hbm.at[pl.ds(base_k + (_NCHUNK - 1) * _GCHUNK, _GCHUNK)],
            wsem)
        writes[_NCHUNK - 2].wait()
        writes[_NCHUNK - 1].wait()
        for i in range(_NCHUNK):
            bias_gathers[i].wait()
        pltpu.sync_copy(biasg_v, biasg_hbm.at[pl.ds(wid * _NCHUNK, _NCHUNK)])

    return gather_kernel(weights, cls, bias)


def _log_corr(cf):
    return jnp.log(_NUM_SAMPLED * jnp.log((cf + 2.0) / (cf + 1.0)) / _LOG_DENOM)


_SCK = 1024


def _loss_body(x_ref, tw_ref, sw_ref, bt_ref, bs_ref, lab_ref, samp_ref, out_ref):
    xb = x_ref[...]
    tw = tw_ref[...]
    labels = lab_ref[...]
    bias_t = bt_ref[...]

    sampled = samp_ref[...]
    lg = lax.dot_general(
        xb.astype(jnp.bfloat16), sw_ref[...].astype(jnp.bfloat16),
        dimension_numbers=(((1,), (1,)), ((), ())),
        preferred_element_type=jnp.float32)
    lg = lg + bs_ref[...] - _log_corr(sampled.astype(jnp.float32))
    lg = jnp.where(labels == sampled, -1e9, lg)

    true_logits = (jnp.sum(xb * tw, axis=1, keepdims=True)
                   + bias_t - _log_corr(labels.astype(jnp.float32)))
    m = jnp.maximum(jnp.max(lg, axis=1, keepdims=True), true_logits)
    sumexp = (jnp.sum(jnp.exp(lg - m), axis=1, keepdims=True)
              + jnp.exp(true_logits - m))
    out_ref[...] = jnp.log(sumexp) + m - true_logits


_BN = 1024


def _tc_loss(x2, w2, bt, bs, lab2, samp2):
    return pl.pallas_call(
        _loss_body,
        grid=(_N // _BN,),
        in_specs=[
            pl.BlockSpec((_BN, _HIDDEN), lambda i: (i, 0)),
            pl.BlockSpec((_BN, _HIDDEN), lambda i: (i, 0)),
            pl.BlockSpec((_S, _HIDDEN), lambda i: (_N // _S, 0)),
            pl.BlockSpec((_BN, 1), lambda i: (i, 0)),
            pl.BlockSpec((1, _S), lambda i: (0, 0)),
            pl.BlockSpec((_BN, 1), lambda i: (i, 0)),
            pl.BlockSpec((1, _S), lambda i: (0, 0)),
        ],
        out_specs=pl.BlockSpec((_BN, 1), lambda i: (i, 0)),
        out_shape=jax.ShapeDtypeStruct((_N, 1), jnp.float32),
    )(x2, w2, w2, bt, bs, lab2, samp2)


def kernel(y_true, input, projection, bias, sampled):
    labels = y_true.reshape(-1)
    x2 = input.reshape(_N, _HIDDEN)
    cls = jnp.concatenate([labels, sampled])
    weights = jnp.swapaxes(projection, 0, 1)
    w2, bias_g = _sc_gather(weights, cls.reshape(_NW, _NCHUNK, _GCHUNK), bias)
    bias_g = bias_g.reshape(-1)
    loss = _tc_loss(x2, w2,
                    bias_g[:_N].reshape(_N, 1), bias_g[_N:].reshape(1, _S),
                    labels.reshape(_N, 1), sampled.reshape(1, _S))
    return loss.reshape(-1)
